# baseline (speedup 1.0000x reference)
import dataclasses
import functools

import jax
import jax.numpy as jnp
from jax.experimental import pallas as pl
from jax.experimental.pallas import tpu as pltpu
from jax.experimental.pallas import tpu_sc as plsc

NBATCH = 4
BLK = 512
NW = 32


def _dot(a, b):
    return jax.lax.dot_general(
        a, b, (((1,), (0,)), ((), ())),
        precision=jax.lax.Precision.HIGHEST,
        preferred_element_type=jnp.float32)


def _proj_body(x_ref, w_ref, b_ref, o_ref):
    o_ref[...] = _dot(x_ref[...], w_ref[...]) + b_ref[...]


def _tc_proj(x2, w, b, blk):
    n, kdim = x2.shape
    h = w.shape[1]
    return pl.pallas_call(
        _proj_body,
        grid=(n // blk,),
        in_specs=[
            pl.BlockSpec((blk, kdim), lambda i: (i, 0)),
            pl.BlockSpec((kdim, h), lambda i: (0, 0)),
            pl.BlockSpec((1, h), lambda i: (0, 0)),
        ],
        out_specs=pl.BlockSpec((blk, h), lambda i: (i, 0)),
        out_shape=jax.ShapeDtypeStruct((n, h), jnp.float32),
    )(x2, w, b.reshape(1, h))


def _excl_cumsum_block(s):
    k = 1
    while k < s.shape[0]:
        s = s + jnp.concatenate(
            [jnp.zeros((k, s.shape[1]), jnp.float32), s[:-k]], axis=0)
        k *= 2
    excl = jnp.concatenate(
        [jnp.zeros((1, s.shape[1]), jnp.float32), s[:-1]], axis=0)
    return excl, s[-1:, :]


def _cumsum_body(h_ref, e_ref, carry):
    i = pl.program_id(0)

    @pl.when(i == 0)
    def _():
        carry[...] = jnp.zeros_like(carry)

    excl, tot = _excl_cumsum_block(h_ref[...])
    c = carry[...]
    e_ref[...] = excl + c
    carry[...] = c + tot


def _tc_cumsum(hmat):
    n, h = hmat.shape
    return pl.pallas_call(
        _cumsum_body,
        grid=(n // BLK,),
        in_specs=[pl.BlockSpec((BLK, h), lambda i: (i, 0))],
        out_specs=pl.BlockSpec((BLK, h), lambda i: (i, 0)),
        out_shape=jax.ShapeDtypeStruct((n, h), jnp.float32),
        scratch_shapes=[pltpu.VMEM((1, h), jnp.float32)],
    )(hmat)


def _leaky(t):
    return jnp.where(t >= 0, t, 0.2 * t)


def _layer_a_body(h_ref, ghi_ref, glo_ref, wr_ref, wn_ref, b_ref, o_ref,
                  e_ref, carry):
    i = pl.program_id(0)

    @pl.when(i == 0)
    def _():
        carry[...] = jnp.zeros_like(carry)

    hv = h_ref[...]
    agg = ghi_ref[...] - glo_ref[...] - hv
    t = _dot(hv, wr_ref[...]) + _dot(agg, wn_ref[...]) + b_ref[...]
    t = _leaky(t)
    o_ref[...] = t
    excl, tot = _excl_cumsum_block(t)
    c = carry[...]
    e_ref[...] = excl + c
    carry[...] = c + tot


def _tc_layer_a(hmat, ghi, glo, wr, wn, b):
    n, h = hmat.shape
    mat = pl.BlockSpec((BLK, h), lambda i: (i, 0))
    wspec = pl.BlockSpec((h, h), lambda i: (0, 0))
    vspec = pl.BlockSpec((1, h), lambda i: (0, 0))
    sds = jax.ShapeDtypeStruct((n, h), jnp.float32)
    return pl.pallas_call(
        _layer_a_body,
        grid=(n // BLK,),
        in_specs=[mat, mat, mat, wspec, wspec, vspec],
        out_specs=(mat, mat),
        out_shape=(sds, sds),
        scratch_shapes=[pltpu.VMEM((1, h), jnp.float32)],
    )(hmat, ghi, glo, wr, wn, b.reshape(1, h))


def _layer_b_body(h_ref, ghi_ref, glo_ref, h0_ref, wr_ref, wn_ref, b_ref,
                  g_ref, beta_ref, o_ref):
    hv = h_ref[...]
    agg = ghi_ref[...] - glo_ref[...] - hv
    t = _dot(hv, wr_ref[...]) + _dot(agg, wn_ref[...]) + b_ref[...]
    mu = jnp.mean(t, axis=-1, keepdims=True)
    var = jnp.mean((t - mu) ** 2, axis=-1, keepdims=True)
    ln = (t - mu) / jnp.sqrt(var + 1e-5) * g_ref[...] + beta_ref[...]
    o_ref[...] = h0_ref[...] + _leaky(ln)


def _tc_layer_b(hmat, ghi, glo, h0, wr, wn, b, g, beta):
    n, h = hmat.shape
    mat = pl.BlockSpec((BLK, h), lambda i: (i, 0))
    wspec = pl.BlockSpec((h, h), lambda i: (0, 0))
    vspec = pl.BlockSpec((1, h), lambda i: (0, 0))
    return pl.pallas_call(
        _layer_b_body,
        grid=(n // BLK,),
        in_specs=[mat, mat, mat, mat, wspec, wspec, vspec, vspec, vspec],
        out_specs=mat,
        out_shape=jax.ShapeDtypeStruct((n, h), jnp.float32),
    )(hmat, ghi, glo, h0, wr, wn, b.reshape(1, h), g.reshape(1, h),
      beta.reshape(1, h))


@functools.cache
def _sc_mesh():
    return plsc.VectorSubcoreMesh(core_axis_name="c", subcore_axis_name="s")


@functools.cache
def _sc_params():
    cp = pltpu.CompilerParams()
    if "needs_layout_passes" in pltpu.CompilerParams.__dataclass_fields__:
        cp = dataclasses.replace(cp, needs_layout_passes=False)
    return cp


def _wid():
    return jax.lax.axis_index("s") * 2 + jax.lax.axis_index("c")


def _sc_depth_prep(ps0, bs0, starts16, cnt, gtab, comp, feat_tab, nd0, rad):
    npad0 = ps0.shape[0]
    npad = comp.shape[0]
    d = feat_tab.shape[1]
    bw = npad // NW
    sds_i = jax.ShapeDtypeStruct((npad,), jnp.int32)

    @functools.partial(
        pl.kernel, mesh=_sc_mesh(),
        out_type=(sds_i, sds_i,
                  jax.ShapeDtypeStruct((npad, d), jnp.float32)),
        compiler_params=_sc_params(),
        scratch_types=[
            pltpu.VMEM((npad0,), jnp.float32),
            pltpu.VMEM((npad0,), jnp.int32),
            pltpu.VMEM((16,), jnp.int32),
            pltpu.VMEM((npad0,), jnp.int32),
            pltpu.VMEM((npad0,), jnp.int32),
            pltpu.VMEM((bw,), jnp.int32),
            pltpu.VMEM((bw,), jnp.int32),
            pltpu.VMEM((bw,), jnp.int32),
            pltpu.VMEM((bw,), jnp.int32),
            pltpu.VMEM((bw, d), jnp.float32),
            pltpu.SemaphoreType.DMA,
        ],
    )
    def k(ps_hbm, bs_hbm, st_hbm, cnt_hbm, g_hbm, comp_hbm, feat_hbm,
          lo_hbm, hi_hbm, rows_hbm,
          ps_v, bs_v, st_v, cnt_v, g_v, comp_v, gidx_v, lo_v, hi_v, rows_v,
          sem):
        base = _wid() * bw
        pltpu.sync_copy(comp_hbm.at[pl.ds(base, bw)], comp_v)
        pltpu.sync_copy(g_hbm, g_v)

        @pl.loop(0, bw, step=16)
        def _(c):
            t = comp_v[pl.ds(c, 16)]
            gidx_v[pl.ds(c, 16)] = plsc.load_gather(g_v, [t])

        cp_rows = pltpu.async_copy(feat_hbm.at[gidx_v], rows_v, sem)

        pltpu.sync_copy(ps_hbm, ps_v)
        pltpu.sync_copy(bs_hbm, bs_v)
        pltpu.sync_copy(st_hbm, st_v)
        pltpu.sync_copy(cnt_hbm, cnt_v)

        @pl.loop(0, bw, step=16)
        def _(c):
            t = comp_v[pl.ds(c, 16)]
            p = plsc.load_gather(ps_v, [t])
            b = plsc.load_gather(bs_v, [t])
            ss = plsc.load_gather(st_v, [b])
            se = plsc.load_gather(st_v, [b + 1])
            lo1, hi1 = ss, se
            lo2, hi2 = ss, se
            for _ in range(14):
                u1 = lo1 < hi1
                m1 = jax.lax.shift_right_logical(lo1 + hi1, 1)
                pm1 = plsc.load_gather(ps_v, [jnp.minimum(m1, nd0 - 1)])
                g1 = (p - pm1) <= rad
                lo1 = jnp.where(u1, jnp.where(g1, lo1, m1 + 1), lo1)
                hi1 = jnp.where(u1, jnp.where(g1, m1, hi1), hi1)
                u2 = lo2 < hi2
                m2 = jax.lax.shift_right_logical(lo2 + hi2, 1)
                pm2 = plsc.load_gather(ps_v, [jnp.minimum(m2, nd0 - 1)])
                g2 = (pm2 - p) > rad
                lo2 = jnp.where(u2, jnp.where(g2, lo2, m2 + 1), lo2)
                hi2 = jnp.where(u2, jnp.where(g2, m2, hi2), hi2)
            lo_v[pl.ds(c, 16)] = plsc.load_gather(cnt_v, [lo1])
            hi_v[pl.ds(c, 16)] = plsc.load_gather(cnt_v, [lo2])

        pltpu.sync_copy(lo_v, lo_hbm.at[pl.ds(base, bw)])
        pltpu.sync_copy(hi_v, hi_hbm.at[pl.ds(base, bw)])
        cp_rows.wait()
        pltpu.sync_copy(rows_v, rows_hbm.at[pl.ds(base, bw)])

    return k(ps0, bs0, starts16, cnt, gtab, comp, feat_tab)


def _sc_gather2(table, idx_hi, idx_lo):
    npad = idx_hi.shape[0]
    d = table.shape[1]
    bw = npad // NW
    sds = jax.ShapeDtypeStruct((npad, d), jnp.float32)

    @functools.partial(
        pl.kernel, mesh=_sc_mesh(),
        out_type=(sds, sds),
        scratch_types=[
            pltpu.VMEM((bw,), jnp.int32),
            pltpu.VMEM((bw,), jnp.int32),
            pltpu.VMEM((bw, d), jnp.float32),
            pltpu.VMEM((bw, d), jnp.float32),
            pltpu.SemaphoreType.DMA,
            pltpu.SemaphoreType.DMA,
        ],
    )
    def k(table_hbm, ih_hbm, il_hbm, ohi_hbm, olo_hbm,
          ih_v, il_v, rh_v, rl_v, sem1, sem2):
        base = _wid() * bw
        pltpu.sync_copy(ih_hbm.at[pl.ds(base, bw)], ih_v)
        pltpu.sync_copy(il_hbm.at[pl.ds(base, bw)], il_v)
        c1 = pltpu.async_copy(table_hbm.at[ih_v], rh_v, sem1)
        c2 = pltpu.async_copy(table_hbm.at[il_v], rl_v, sem2)
        c1.wait()
        c2.wait()
        pltpu.sync_copy(rh_v, ohi_hbm.at[pl.ds(base, bw)])
        pltpu.sync_copy(rl_v, olo_hbm.at[pl.ds(base, bw)])

    return k(table, idx_hi, idx_lo)


def _sc_gather(feat_tab, src_idx):
    npad = src_idx.shape[0]
    d = feat_tab.shape[1]
    bw = npad // NW

    @functools.partial(
        pl.kernel, mesh=_sc_mesh(),
        out_type=jax.ShapeDtypeStruct((npad, d), jnp.float32),
        scratch_types=[
            pltpu.VMEM((bw,), jnp.int32),
            pltpu.VMEM((bw, d), jnp.float32),
            pltpu.SemaphoreType.DMA,
        ],
    )
    def k(feat_hbm, idx_hbm, out_hbm, idx_v, rows_v, sem):
        base = _wid() * bw
        pltpu.sync_copy(idx_hbm.at[pl.ds(base, bw)], idx_v)
        pltpu.async_copy(feat_hbm.at[idx_v], rows_v, sem).wait()
        pltpu.sync_copy(rows_v, out_hbm.at[pl.ds(base, bw)])

    return k(feat_tab, src_idx)


def kernel(x, pos, batch, mask, indices, proj_W, proj_b, Wroot, Wneigh,
           conv_b, ln_g, ln_b):
    n0 = x.shape[0]
    depth = Wroot.shape[0]
    nl = Wroot.shape[1]
    p0 = pos[:, 0]

    npads = {d: ((n0 >> d) + BLK) // BLK * BLK for d in range(depth)}
    npad0 = npads[0]

    feat0 = _tc_proj(x.reshape(n0, -1), proj_W, proj_b, 400)

    npad_tail = npad0 - n0
    batch_in = jnp.concatenate(
        [batch.astype(jnp.int32), jnp.full((npad_tail,), NBATCH, jnp.int32)])
    p_in = jnp.concatenate([p0, jnp.zeros((npad_tail,), jnp.float32)])
    ids_in = jnp.concatenate(
        [jnp.arange(n0, dtype=jnp.int32), jnp.zeros((npad_tail,), jnp.int32)])
    bs0_pad, ps0_pad, perm0_pad = jax.lax.sort(
        (batch_in, p_in, ids_in), num_keys=2)

    counts = jnp.sum((bs0_pad[:n0, None] == jnp.arange(NBATCH)[None, :])
                     .astype(jnp.int32), axis=0)
    starts = jnp.concatenate(
        [jnp.zeros((1,), jnp.int32), jnp.cumsum(counts)]).astype(jnp.int32)
    starts16 = jnp.full((16,), n0, jnp.int32).at[:NBATCH + 1].set(starts)

    iota0 = jnp.arange(npad0, dtype=jnp.int32)
    comp0 = jnp.where(iota0 < n0, iota0, 0)
    valid0 = iota0 < n0

    def cnt_of(step):
        act = ((perm0_pad % step) == 0) & valid0
        return jnp.concatenate(
            [jnp.zeros((1,), jnp.int32),
             jnp.cumsum(act.astype(jnp.int32))])[:npad0], act

    feat_s = None
    comp = comp0
    gtab = perm0_pad
    cnt_pad = iota0
    cnt_prev = None
    for d in range(depth):
        rad = float(2.0 * (1 << d))
        src = feat0 if d == 0 else feat_s
        lo_p, hi_p, feat_s = _sc_depth_prep(
            ps0_pad, bs0_pad, starts16, cnt_pad, gtab, comp, src, n0, rad)

        hcur = feat_s
        e_mat = _tc_cumsum(hcur)
        for l in range(nl):
            ghi, glo = _sc_gather2(e_mat, hi_p, lo_p)
            if l < nl - 1:
                hcur, e_mat = _tc_layer_a(hcur, ghi, glo, Wroot[d, l],
                                          Wneigh[d, l], conv_b[d, l])
            else:
                feat_s = _tc_layer_b(hcur, ghi, glo, feat_s, Wroot[d, l],
                                     Wneigh[d, l], conv_b[d, l], ln_g[d],
                                     ln_b[d])

        cnt_prev = cnt_pad
        cnt_pad, act = cnt_of(1 << (d + 1))
        if d + 1 < depth:
            comp = jnp.nonzero(act, size=npads[d + 1],
                               fill_value=0)[0].astype(jnp.int32)
            gtab = cnt_prev

    nout = n0 >> depth
    npad_out = (nout + 255) // 256 * 256
    _, act3 = cnt_of(1 << depth)
    comp3 = jnp.nonzero(act3, size=nout, fill_value=0)[0].astype(jnp.int32)
    out_idx = jnp.zeros((npad_out,), jnp.int32).at[
        perm0_pad[comp3] >> depth].set(cnt_prev[comp3])
    out = _sc_gather(feat_s, out_idx)
    return out[:nout]


# speedup vs baseline: 1.0052x; 1.0052x over previous
import dataclasses
import functools

import jax
import jax.numpy as jnp
from jax.experimental import pallas as pl
from jax.experimental.pallas import tpu as pltpu
from jax.experimental.pallas import tpu_sc as plsc

NBATCH = 4
BLK = 512
NW = 32


def _dot(a, b):
    return jax.lax.dot_general(
        a, b, (((1,), (0,)), ((), ())),
        precision=jax.lax.Precision.HIGHEST,
        preferred_element_type=jnp.float32)


def _proj_body(x_ref, w_ref, b_ref, o_ref):
    o_ref[...] = _dot(x_ref[...], w_ref[...]) + b_ref[...]


def _tc_proj(x2, w, b, blk):
    n, kdim = x2.shape
    h = w.shape[1]
    return pl.pallas_call(
        _proj_body,
        grid=(n // blk,),
        in_specs=[
            pl.BlockSpec((blk, kdim), lambda i: (i, 0)),
            pl.BlockSpec((kdim, h), lambda i: (0, 0)),
            pl.BlockSpec((1, h), lambda i: (0, 0)),
        ],
        out_specs=pl.BlockSpec((blk, h), lambda i: (i, 0)),
        out_shape=jax.ShapeDtypeStruct((n, h), jnp.float32),
    )(x2, w, b.reshape(1, h))


def _excl_cumsum_block(s):
    k = 1
    while k < s.shape[0]:
        s = s + jnp.concatenate(
            [jnp.zeros((k, s.shape[1]), jnp.float32), s[:-k]], axis=0)
        k *= 2
    excl = jnp.concatenate(
        [jnp.zeros((1, s.shape[1]), jnp.float32), s[:-1]], axis=0)
    return excl, s[-1:, :]


def _cumsum_body(h_ref, e_ref, carry):
    i = pl.program_id(0)

    @pl.when(i == 0)
    def _():
        carry[...] = jnp.zeros_like(carry)

    excl, tot = _excl_cumsum_block(h_ref[...])
    c = carry[...]
    e_ref[...] = excl + c
    carry[...] = c + tot


def _tc_cumsum(hmat):
    n, h = hmat.shape
    return pl.pallas_call(
        _cumsum_body,
        grid=(n // BLK,),
        in_specs=[pl.BlockSpec((BLK, h), lambda i: (i, 0))],
        out_specs=pl.BlockSpec((BLK, h), lambda i: (i, 0)),
        out_shape=jax.ShapeDtypeStruct((n, h), jnp.float32),
        scratch_shapes=[pltpu.VMEM((1, h), jnp.float32)],
    )(hmat)


def _leaky(t):
    return jnp.where(t >= 0, t, 0.2 * t)


def _layer_a_body(h_ref, ghi_ref, glo_ref, wr_ref, wn_ref, b_ref, o_ref,
                  e_ref, carry):
    i = pl.program_id(0)

    @pl.when(i == 0)
    def _():
        carry[...] = jnp.zeros_like(carry)

    hv = h_ref[...]
    agg = ghi_ref[...] - glo_ref[...] - hv
    t = _dot(hv, wr_ref[...]) + _dot(agg, wn_ref[...]) + b_ref[...]
    t = _leaky(t)
    o_ref[...] = t
    excl, tot = _excl_cumsum_block(t)
    c = carry[...]
    e_ref[...] = excl + c
    carry[...] = c + tot


def _tc_layer_a(hmat, ghi, glo, wr, wn, b):
    n, h = hmat.shape
    mat = pl.BlockSpec((BLK, h), lambda i: (i, 0))
    wspec = pl.BlockSpec((h, h), lambda i: (0, 0))
    vspec = pl.BlockSpec((1, h), lambda i: (0, 0))
    sds = jax.ShapeDtypeStruct((n, h), jnp.float32)
    return pl.pallas_call(
        _layer_a_body,
        grid=(n // BLK,),
        in_specs=[mat, mat, mat, wspec, wspec, vspec],
        out_specs=(mat, mat),
        out_shape=(sds, sds),
        scratch_shapes=[pltpu.VMEM((1, h), jnp.float32)],
    )(hmat, ghi, glo, wr, wn, b.reshape(1, h))


def _layer_b_body(h_ref, ghi_ref, glo_ref, h0_ref, wr_ref, wn_ref, b_ref,
                  g_ref, beta_ref, o_ref):
    hv = h_ref[...]
    agg = ghi_ref[...] - glo_ref[...] - hv
    t = _dot(hv, wr_ref[...]) + _dot(agg, wn_ref[...]) + b_ref[...]
    mu = jnp.mean(t, axis=-1, keepdims=True)
    var = jnp.mean((t - mu) ** 2, axis=-1, keepdims=True)
    ln = (t - mu) / jnp.sqrt(var + 1e-5) * g_ref[...] + beta_ref[...]
    o_ref[...] = h0_ref[...] + _leaky(ln)


def _tc_layer_b(hmat, ghi, glo, h0, wr, wn, b, g, beta):
    n, h = hmat.shape
    mat = pl.BlockSpec((BLK, h), lambda i: (i, 0))
    wspec = pl.BlockSpec((h, h), lambda i: (0, 0))
    vspec = pl.BlockSpec((1, h), lambda i: (0, 0))
    return pl.pallas_call(
        _layer_b_body,
        grid=(n // BLK,),
        in_specs=[mat, mat, mat, mat, wspec, wspec, vspec, vspec, vspec],
        out_specs=mat,
        out_shape=jax.ShapeDtypeStruct((n, h), jnp.float32),
    )(hmat, ghi, glo, h0, wr, wn, b.reshape(1, h), g.reshape(1, h),
      beta.reshape(1, h))


@functools.cache
def _sc_mesh():
    return plsc.VectorSubcoreMesh(core_axis_name="c", subcore_axis_name="s")


@functools.cache
def _sc_params():
    cp = pltpu.CompilerParams()
    if "needs_layout_passes" in pltpu.CompilerParams.__dataclass_fields__:
        cp = dataclasses.replace(cp, needs_layout_passes=False)
    return cp


def _wid():
    return jax.lax.axis_index("s") * 2 + jax.lax.axis_index("c")


def _sc_depth_prep(ps0, bs0, starts16, cnt, gtab, comp, feat_tab, nd0, rad):
    npad0 = ps0.shape[0]
    npad = comp.shape[0]
    d = feat_tab.shape[1]
    bw = npad // NW
    sds_i = jax.ShapeDtypeStruct((npad,), jnp.int32)

    @functools.partial(
        pl.kernel, mesh=_sc_mesh(),
        out_type=(sds_i, sds_i,
                  jax.ShapeDtypeStruct((npad, d), jnp.float32)),
        compiler_params=_sc_params(),
        scratch_types=[
            pltpu.VMEM((npad0,), jnp.float32),
            pltpu.VMEM((npad0,), jnp.int32),
            pltpu.VMEM((16,), jnp.int32),
            pltpu.VMEM((npad0,), jnp.int32),
            pltpu.VMEM((npad0,), jnp.int32),
            pltpu.VMEM((bw,), jnp.int32),
            pltpu.VMEM((bw,), jnp.int32),
            pltpu.VMEM((bw,), jnp.int32),
            pltpu.VMEM((bw,), jnp.int32),
            pltpu.VMEM((bw, d), jnp.float32),
            pltpu.SemaphoreType.DMA,
        ],
    )
    def k(ps_hbm, bs_hbm, st_hbm, cnt_hbm, g_hbm, comp_hbm, feat_hbm,
          lo_hbm, hi_hbm, rows_hbm,
          ps_v, bs_v, st_v, cnt_v, g_v, comp_v, gidx_v, lo_v, hi_v, rows_v,
          sem):
        base = _wid() * bw
        pltpu.sync_copy(comp_hbm.at[pl.ds(base, bw)], comp_v)
        pltpu.sync_copy(g_hbm, g_v)

        @pl.loop(0, bw, step=16)
        def _(c):
            t = comp_v[pl.ds(c, 16)]
            gidx_v[pl.ds(c, 16)] = plsc.load_gather(g_v, [t])

        cp_rows = pltpu.async_copy(feat_hbm.at[gidx_v], rows_v, sem)

        pltpu.sync_copy(ps_hbm, ps_v)
        pltpu.sync_copy(bs_hbm, bs_v)
        pltpu.sync_copy(st_hbm, st_v)
        pltpu.sync_copy(cnt_hbm, cnt_v)

        @pl.loop(0, bw, step=16)
        def _(c):
            t = comp_v[pl.ds(c, 16)]
            p = plsc.load_gather(ps_v, [t])
            b = plsc.load_gather(bs_v, [t])
            ss = plsc.load_gather(st_v, [b])
            se = plsc.load_gather(st_v, [b + 1])
            lo1, hi1 = ss, se
            lo2, hi2 = ss, se
            for _ in range(14):
                u1 = lo1 < hi1
                m1 = jax.lax.shift_right_logical(lo1 + hi1, 1)
                pm1 = plsc.load_gather(ps_v, [jnp.minimum(m1, nd0 - 1)])
                g1 = (p - pm1) <= rad
                lo1 = jnp.where(u1, jnp.where(g1, lo1, m1 + 1), lo1)
                hi1 = jnp.where(u1, jnp.where(g1, m1, hi1), hi1)
                u2 = lo2 < hi2
                m2 = jax.lax.shift_right_logical(lo2 + hi2, 1)
                pm2 = plsc.load_gather(ps_v, [jnp.minimum(m2, nd0 - 1)])
                g2 = (pm2 - p) > rad
                lo2 = jnp.where(u2, jnp.where(g2, lo2, m2 + 1), lo2)
                hi2 = jnp.where(u2, jnp.where(g2, m2, hi2), hi2)
            lo_v[pl.ds(c, 16)] = plsc.load_gather(cnt_v, [lo1])
            hi_v[pl.ds(c, 16)] = plsc.load_gather(cnt_v, [lo2])

        pltpu.sync_copy(lo_v, lo_hbm.at[pl.ds(base, bw)])
        pltpu.sync_copy(hi_v, hi_hbm.at[pl.ds(base, bw)])
        cp_rows.wait()
        pltpu.sync_copy(rows_v, rows_hbm.at[pl.ds(base, bw)])

    return k(ps0, bs0, starts16, cnt, gtab, comp, feat_tab)


def _sc_gather2(table, idx_hi, idx_lo):
    npad = idx_hi.shape[0]
    d = table.shape[1]
    bw = npad // NW
    sds = jax.ShapeDtypeStruct((npad, d), jnp.float32)

    @functools.partial(
        pl.kernel, mesh=_sc_mesh(),
        out_type=(sds, sds),
        scratch_types=[
            pltpu.VMEM((bw,), jnp.int32),
            pltpu.VMEM((bw,), jnp.int32),
            pltpu.VMEM((bw, d), jnp.float32),
            pltpu.VMEM((bw, d), jnp.float32),
            pltpu.SemaphoreType.DMA,
            pltpu.SemaphoreType.DMA,
        ],
    )
    def k(table_hbm, ih_hbm, il_hbm, ohi_hbm, olo_hbm,
          ih_v, il_v, rh_v, rl_v, sem1, sem2):
        base = _wid() * bw
        pltpu.sync_copy(ih_hbm.at[pl.ds(base, bw)], ih_v)
        pltpu.sync_copy(il_hbm.at[pl.ds(base, bw)], il_v)
        c1 = pltpu.async_copy(table_hbm.at[ih_v], rh_v, sem1)
        c2 = pltpu.async_copy(table_hbm.at[il_v], rl_v, sem2)
        c1.wait()
        c2.wait()
        pltpu.sync_copy(rh_v, ohi_hbm.at[pl.ds(base, bw)])
        pltpu.sync_copy(rl_v, olo_hbm.at[pl.ds(base, bw)])

    return k(table, idx_hi, idx_lo)


def _sc_gather(feat_tab, src_idx):
    npad = src_idx.shape[0]
    d = feat_tab.shape[1]
    bw = npad // NW

    @functools.partial(
        pl.kernel, mesh=_sc_mesh(),
        out_type=jax.ShapeDtypeStruct((npad, d), jnp.float32),
        scratch_types=[
            pltpu.VMEM((bw,), jnp.int32),
            pltpu.VMEM((bw, d), jnp.float32),
            pltpu.SemaphoreType.DMA,
        ],
    )
    def k(feat_hbm, idx_hbm, out_hbm, idx_v, rows_v, sem):
        base = _wid() * bw
        pltpu.sync_copy(idx_hbm.at[pl.ds(base, bw)], idx_v)
        pltpu.async_copy(feat_hbm.at[idx_v], rows_v, sem).wait()
        pltpu.sync_copy(rows_v, out_hbm.at[pl.ds(base, bw)])

    return k(feat_tab, src_idx)


def kernel(x, pos, batch, mask, indices, proj_W, proj_b, Wroot, Wneigh,
           conv_b, ln_g, ln_b):
    n0 = x.shape[0]
    depth = Wroot.shape[0]
    nl = Wroot.shape[1]
    p0 = pos[:, 0]

    npads = {d: ((n0 >> d) + BLK) // BLK * BLK for d in range(depth)}
    npad0 = npads[0]

    feat0 = _tc_proj(x.reshape(n0, -1), proj_W, proj_b, 400)

    bs0, ps0, perm0 = jax.lax.sort(
        (batch.astype(jnp.int32), p0, jnp.arange(n0, dtype=jnp.int32)),
        num_keys=2)

    counts = jnp.sum((bs0[:, None] == jnp.arange(NBATCH)[None, :])
                     .astype(jnp.int32), axis=0)
    starts = jnp.concatenate(
        [jnp.zeros((1,), jnp.int32), jnp.cumsum(counts)]).astype(jnp.int32)
    starts16 = jnp.full((16,), n0, jnp.int32).at[:NBATCH + 1].set(starts)

    ps0_pad = jnp.zeros((npad0,), jnp.float32).at[:n0].set(ps0)
    bs0_pad = jnp.full((npad0,), NBATCH, jnp.int32).at[:n0].set(bs0)
    perm0_pad = jnp.zeros((npad0,), jnp.int32).at[:n0].set(perm0)
    iota0 = jnp.arange(npad0, dtype=jnp.int32)
    comp0 = jnp.where(iota0 < n0, iota0, 0)
    valid0 = iota0 < n0

    def cnt_of(step):
        act = ((perm0_pad % step) == 0) & valid0
        return jnp.concatenate(
            [jnp.zeros((1,), jnp.int32),
             jnp.cumsum(act.astype(jnp.int32))])[:npad0], act

    feat_s = None
    comp = comp0
    gtab = perm0_pad
    cnt_pad = iota0
    cnt_prev = None
    for d in range(depth):
        rad = float(2.0 * (1 << d))
        src = feat0 if d == 0 else feat_s
        lo_p, hi_p, feat_s = _sc_depth_prep(
            ps0_pad, bs0_pad, starts16, cnt_pad, gtab, comp, src, n0, rad)

        hcur = feat_s
        e_mat = _tc_cumsum(hcur)
        for l in range(nl):
            ghi, glo = _sc_gather2(e_mat, hi_p, lo_p)
            if l < nl - 1:
                hcur, e_mat = _tc_layer_a(hcur, ghi, glo, Wroot[d, l],
                                          Wneigh[d, l], conv_b[d, l])
            else:
                feat_s = _tc_layer_b(hcur, ghi, glo, feat_s, Wroot[d, l],
                                     Wneigh[d, l], conv_b[d, l], ln_g[d],
                                     ln_b[d])

        cnt_prev = cnt_pad
        cnt_pad, act = cnt_of(1 << (d + 1))
        if d + 1 < depth:
            comp = jnp.nonzero(act, size=npads[d + 1],
                               fill_value=0)[0].astype(jnp.int32)
            gtab = cnt_prev

    nout = n0 >> depth
    npad_out = (nout + 255) // 256 * 256
    _, act3 = cnt_of(1 << depth)
    comp3 = jnp.nonzero(act3, size=nout, fill_value=0)[0].astype(jnp.int32)
    out_idx = jnp.zeros((npad_out,), jnp.int32).at[
        perm0_pad[comp3] >> depth].set(cnt_prev[comp3])
    out = _sc_gather(feat_s, out_idx)
    return out[:nout]
